# baseline (speedup 1.0000x reference)
import functools

import jax
import jax.numpy as jnp
from jax import lax
from jax.experimental import pallas as pl
from jax.experimental.pallas import tpu as pltpu
from jax.experimental.pallas import tpu_sc as plsc


def _route_body(x_ref, wg_ref, bg_ref, slot_ref, te_ref, oe_ref, psum_ref,
                loss_ref, carry_ref, ke_ref, cb0_ref, tm_ref, carry2_ref,
                rank2_ref, off_ref, *, nexp, tb, cap, total, blk, nslot,
                ntile):
    i = pl.program_id(0)

    @pl.when(i == 0)
    def _():
        carry_ref[...] = jnp.zeros_like(carry_ref)
        psum_ref[...] = jnp.zeros_like(psum_ref)

    lane8 = jax.lax.broadcasted_iota(jnp.int32, (1, nexp), 1)
    rr = jax.lax.broadcasted_iota(jnp.int32, (nexp, nexp), 0)
    cc = jax.lax.broadcasted_iota(jnp.int32, (nexp, nexp), 1)
    u8 = (rr < cc).astype(jnp.float32)
    r2 = jax.lax.broadcasted_iota(jnp.int32, (tb, tb), 0)
    c2 = jax.lax.broadcasted_iota(jnp.int32, (tb, tb), 1)
    ltri = (r2 > c2).astype(jnp.float32)
    riota = jax.lax.broadcasted_iota(jnp.int32, (tb, 1), 0)

    @pl.when(i < 8)
    def _():
        x = x_ref[...]
        logits = jnp.dot(x, wg_ref[...], preferred_element_type=jnp.float32)
        logits = logits + bg_ref[...]
        m = jnp.max(logits, axis=-1, keepdims=True)
        p = jnp.exp(logits - m)
        probs = p / jnp.sum(p, axis=-1, keepdims=True)
        psum_ref[...] += jnp.sum(probs, axis=0, keepdims=True)

        is_max = (logits == m).astype(jnp.float32)
        prior = jnp.dot(is_max, u8, preferred_element_type=jnp.float32)
        onehot = is_max * (prior == 0).astype(jnp.float32)

        rank = jnp.dot(ltri, onehot, preferred_element_type=jnp.float32)
        rank = rank + carry_ref[...]
        kept = (onehot > 0.5) & (rank < float(cap))
        lane = jax.lax.broadcasted_iota(jnp.int32, (tb, nexp), 1)
        ke = jnp.max(jnp.where(kept, lane, -1), axis=-1, keepdims=True)
        carry_ref[...] += jnp.sum(onehot, axis=0, keepdims=True)

        for k in range(4):
            @pl.when(i == k)
            def _(k=k):
                ke_ref[:, k:k + 1] = ke

            @pl.when(i == k + 4)
            def _(k=k):
                tm_ref[k * tb:(k + 1) * tb, :] = jnp.maximum(
                    ke_ref[:, k:k + 1], ke)

        @pl.when(i == 3)
        def _():
            cb0_ref[...] = carry_ref[...]

        @pl.when(i == 7)
        def _():
            loss_ref[...] = (jnp.sum(psum_ref[...] * carry_ref[...],
                                     axis=(0, 1), keepdims=True)
                             * (nexp / float(total) / float(total)))

    @pl.when(i == 8)
    def _():
        cb0 = cb0_ref[...]
        ctot = carry_ref[...]
        a0 = jnp.max(jnp.where(cb0 > 0.5, lane8, -1))
        a1 = jnp.max(jnp.where((ctot - cb0 > 0.5) & (cb0 < float(cap)),
                               lane8, -1))
        slab = tm_ref[0:tb, :]
        slab = jnp.where(riota == 0, jnp.maximum(slab, a0), slab)
        slab = jnp.where(riota == 1, jnp.maximum(slab, a1), slab)
        tm_ref[0:tb, :] = slab
        carry2_ref[...] = jnp.zeros_like(carry2_ref)

    @pl.when((i >= 8) & (i < 12))
    def _():
        for k in range(4):
            @pl.when(i == k + 8)
            def _(k=k):
                tmv = tm_ref[k * tb:(k + 1) * tb, :]
                onehot2 = (tmv == lane8).astype(jnp.float32)
                rank = jnp.dot(ltri, onehot2,
                               preferred_element_type=jnp.float32)
                rank = rank + carry2_ref[...]
                rank2_ref[:, k:k + 1] = jnp.sum(
                    onehot2 * rank, axis=-1, keepdims=True)
                carry2_ref[...] += jnp.sum(onehot2, axis=0, keepdims=True)

    @pl.when(i == 11)
    def _():
        cnt2 = carry2_ref[...]
        padded = jnp.ceil(cnt2 / blk) * blk
        offv = jnp.dot(padded, u8, preferred_element_type=jnp.float32)
        off_ref[...] = offv
        tcol = jax.lax.broadcasted_iota(jnp.int32, (ntile, 1), 0)
        ge = (tcol.astype(jnp.float32) >= offv / blk).astype(jnp.int32)
        te = jnp.sum(ge, axis=-1, keepdims=True) - 1
        oec = offv + cnt2
        oe = jnp.sum((te == lane8).astype(jnp.float32) * oec,
                     axis=-1, keepdims=True)
        te_ref[...] = te
        oe_ref[...] = oe.astype(jnp.int32)

    @pl.when(i >= 12)
    def _():
        offv = off_ref[...]
        for k in range(4):
            @pl.when(i == k + 12)
            def _(k=k):
                tmv = tm_ref[k * tb:(k + 1) * tb, :]
                onehot2 = (tmv == lane8).astype(jnp.float32)
                off_tok = jnp.sum(onehot2 * offv, axis=-1, keepdims=True)
                any_e = jnp.sum(onehot2, axis=-1, keepdims=True) > 0.5
                slot = off_tok + rank2_ref[:, k:k + 1]
                slot_ref[0] = jnp.where(any_e, slot.astype(jnp.int32),
                                        nslot - 1)


def _dispatch_body(x_hbm, slot_hbm, xs_hbm, idx_v, rows_v, sem, *, nc, rw):
    wid = lax.axis_index("s") * nc + lax.axis_index("c")
    pltpu.sync_copy(slot_hbm.at[wid], idx_v)
    pltpu.sync_copy(x_hbm.at[pl.ds(wid * rw, rw)], rows_v)
    pltpu.async_copy(rows_v, xs_hbm.at[idx_v], sem).wait()


def _combine_body(ys_hbm, slot_hbm, y0_hbm, idx_v, rows_v, sem, *, nc, rw):
    wid = lax.axis_index("s") * nc + lax.axis_index("c")
    pltpu.sync_copy(slot_hbm.at[wid], idx_v)
    pltpu.async_copy(ys_hbm.at[idx_v], rows_v, sem).wait()
    pltpu.sync_copy(rows_v, y0_hbm.at[pl.ds(wid * rw, rw)])


def _gffn_body(te_ref, oe_ref, x_ref, w1_ref, b1_ref, w2_ref, b2_ref, y_ref,
               *, blk):
    t = pl.program_id(0)
    x = x_ref[...].astype(jnp.bfloat16)
    h = jnp.dot(x, w1_ref[0], preferred_element_type=jnp.float32)
    h = jnp.maximum(h + b1_ref[0], 0.0)
    y = jnp.dot(h.astype(jnp.bfloat16), w2_ref[0],
                preferred_element_type=jnp.float32)
    y = y + b2_ref[0]
    row = jax.lax.broadcasted_iota(jnp.int32, y.shape, 0) + t * blk
    y_ref[...] = jnp.where(row < oe_ref[t], y, 0.0)


def kernel(x, Wg, bg, W1, b1, W2, b2):
    b, s, d = x.shape
    nexp = Wg.shape[1]
    total = b * s
    cap = int(total / nexp * 1.25)
    nt = 8
    tb = total // nt
    blk = 128
    nslot = s + nexp * blk
    ntile = nslot // blk
    xf = x.reshape(total, d)

    route = functools.partial(
        _route_body, nexp=nexp, tb=tb, cap=cap, total=total, blk=blk,
        nslot=nslot, ntile=ntile)
    slot4, te, oe, psum, loss = pl.pallas_call(
        route,
        grid=(2 * nt,),
        in_specs=[
            pl.BlockSpec((tb, d), lambda i: (jnp.minimum(i, 7), 0)),
            pl.BlockSpec((d, nexp), lambda i: (0, 0)),
            pl.BlockSpec((1, nexp), lambda i: (0, 0)),
        ],
        out_specs=[
            pl.BlockSpec((1, tb, 1),
                         lambda i: (jnp.maximum(i - 12, 0), 0, 0)),
            pl.BlockSpec((ntile, 1), lambda i: (0, 0)),
            pl.BlockSpec((ntile, 1), lambda i: (0, 0)),
            pl.BlockSpec((1, nexp), lambda i: (0, 0)),
            pl.BlockSpec((1, 1), lambda i: (0, 0)),
        ],
        out_shape=[
            jax.ShapeDtypeStruct((4, tb, 1), jnp.int32),
            jax.ShapeDtypeStruct((ntile, 1), jnp.int32),
            jax.ShapeDtypeStruct((ntile, 1), jnp.int32),
            jax.ShapeDtypeStruct((1, nexp), jnp.float32),
            jax.ShapeDtypeStruct((1, 1), jnp.float32),
        ],
        scratch_shapes=[
            pltpu.VMEM((1, nexp), jnp.float32),
            pltpu.VMEM((tb, 4), jnp.int32),
            pltpu.VMEM((1, nexp), jnp.float32),
            pltpu.VMEM((s, 1), jnp.int32),
            pltpu.VMEM((1, nexp), jnp.float32),
            pltpu.VMEM((tb, 4), jnp.float32),
            pltpu.VMEM((1, nexp), jnp.float32),
        ],
    )(xf, Wg, bg.reshape(1, nexp))

    info = plsc.get_sparse_core_info()
    nc, ns = info.num_cores, info.num_subcores
    nw = nc * ns
    rw = s // nw
    slot_w = slot4.reshape(nw, rw)
    mesh = plsc.VectorSubcoreMesh(core_axis_name="c", subcore_axis_name="s")

    xs = pl.kernel(
        functools.partial(_dispatch_body, nc=nc, rw=rw),
        out_type=jax.ShapeDtypeStruct((nslot, d), jnp.float32),
        mesh=mesh,
        scratch_types=[
            pltpu.VMEM((rw,), jnp.int32),
            pltpu.VMEM((rw, d), jnp.float32),
            pltpu.SemaphoreType.DMA,
        ],
    )(x[0], slot_w)

    gffn = functools.partial(_gffn_body, blk=blk)
    ys = pl.pallas_call(
        gffn,
        grid_spec=pltpu.PrefetchScalarGridSpec(
            num_scalar_prefetch=2,
            grid=(ntile,),
            in_specs=[
                pl.BlockSpec((blk, d), lambda t, te_r, oe_r: (t, 0)),
                pl.BlockSpec((1, d, d),
                             lambda t, te_r, oe_r: (te_r[t], 0, 0)),
                pl.BlockSpec((1, 1, d),
                             lambda t, te_r, oe_r: (te_r[t], 0, 0)),
                pl.BlockSpec((1, d, d),
                             lambda t, te_r, oe_r: (te_r[t], 0, 0)),
                pl.BlockSpec((1, 1, d),
                             lambda t, te_r, oe_r: (te_r[t], 0, 0)),
            ],
            out_specs=pl.BlockSpec((blk, d), lambda t, te_r, oe_r: (t, 0)),
        ),
        out_shape=jax.ShapeDtypeStruct((nslot, d), jnp.float32),
    )(te.reshape(ntile), oe.reshape(ntile), xs,
      W1.astype(jnp.bfloat16), b1.reshape(nexp, 1, d),
      W2.astype(jnp.bfloat16), b2.reshape(nexp, 1, d))

    y0 = pl.kernel(
        functools.partial(_combine_body, nc=nc, rw=rw),
        out_type=jax.ShapeDtypeStruct((s, d), jnp.float32),
        mesh=mesh,
        scratch_types=[
            pltpu.VMEM((rw,), jnp.int32),
            pltpu.VMEM((rw, d), jnp.float32),
            pltpu.SemaphoreType.DMA,
        ],
    )(ys, slot_w)

    out = jnp.concatenate([y0[None], jnp.zeros_like(y0)[None]], axis=0)
    return out, loss[0, 0]


# speedup vs baseline: 1.2034x; 1.2034x over previous
import functools

import jax
import jax.numpy as jnp
from jax import lax
from jax.experimental import pallas as pl
from jax.experimental.pallas import tpu as pltpu
from jax.experimental.pallas import tpu_sc as plsc


def _route_body(x_ref, wg_ref, bg_ref, slot_ref, te_ref, oe_ref, psum_ref,
                loss_ref, carry_ref, ke_ref, cb0_ref, tm_ref,
                carry2_ref, rank2_ref, off_ref, *, nexp, tb, cap, total, blk,
                nslot, ntile):
    i = pl.program_id(0)

    @pl.when(i == 0)
    def _():
        carry_ref[...] = jnp.zeros_like(carry_ref)
        psum_ref[...] = jnp.zeros_like(psum_ref)

    lane8 = jax.lax.broadcasted_iota(jnp.int32, (1, nexp), 1)
    rr = jax.lax.broadcasted_iota(jnp.int32, (nexp, nexp), 0)
    cc = jax.lax.broadcasted_iota(jnp.int32, (nexp, nexp), 1)
    u8 = (rr < cc).astype(jnp.float32)
    r2 = jax.lax.broadcasted_iota(jnp.int32, (tb, tb), 0)
    c2 = jax.lax.broadcasted_iota(jnp.int32, (tb, tb), 1)
    ltri = (r2 > c2).astype(jnp.float32)
    riota = jax.lax.broadcasted_iota(jnp.int32, (tb, 1), 0)

    @pl.when(i < 8)
    def _():
        x = x_ref[...]
        logits = jnp.dot(x, wg_ref[...], preferred_element_type=jnp.float32)
        logits = logits + bg_ref[...]
        m = jnp.max(logits, axis=-1, keepdims=True)
        p = jnp.exp(logits - m)
        probs = p / jnp.sum(p, axis=-1, keepdims=True)
        psum_ref[...] += jnp.sum(probs, axis=0, keepdims=True)

        is_max = (logits == m).astype(jnp.float32)
        prior = jnp.dot(is_max, u8, preferred_element_type=jnp.float32)
        onehot = is_max * (prior == 0).astype(jnp.float32)

        rank = jnp.dot(ltri, onehot, preferred_element_type=jnp.float32)
        rank = rank + carry_ref[...]
        kept = (onehot > 0.5) & (rank < float(cap))
        lane = jax.lax.broadcasted_iota(jnp.int32, (tb, nexp), 1)
        ke = jnp.max(jnp.where(kept, lane, -1), axis=-1, keepdims=True)
        carry_ref[...] += jnp.sum(onehot, axis=0, keepdims=True)

        for k in range(4):
            @pl.when(i == k)
            def _(k=k):
                ke_ref[:, k:k + 1] = ke

            @pl.when(i == k + 4)
            def _(k=k):
                tm_ref[k * tb:(k + 1) * tb, :] = jnp.maximum(
                    ke_ref[:, k:k + 1], ke)

        @pl.when(i == 3)
        def _():
            cb0_ref[...] = carry_ref[...]

        @pl.when(i == 7)
        def _():
            loss_ref[...] = (jnp.sum(psum_ref[...] * carry_ref[...],
                                     axis=(0, 1), keepdims=True)
                             * (nexp / float(total) / float(total)))

    @pl.when(i == 8)
    def _():
        cb0 = cb0_ref[...]
        ctot = carry_ref[...]
        a0 = jnp.max(jnp.where(cb0 > 0.5, lane8, -1))
        a1 = jnp.max(jnp.where((ctot - cb0 > 0.5) & (cb0 < float(cap)),
                               lane8, -1))
        slab = tm_ref[0:tb, :]
        slab = jnp.where(riota == 0, jnp.maximum(slab, a0), slab)
        slab = jnp.where(riota == 1, jnp.maximum(slab, a1), slab)
        tm_ref[0:tb, :] = slab
        carry2_ref[...] = jnp.zeros_like(carry2_ref)

    @pl.when((i >= 8) & (i < 12))
    def _():
        for k in range(4):
            @pl.when(i == k + 8)
            def _(k=k):
                tmv = tm_ref[k * tb:(k + 1) * tb, :]
                onehot2 = (tmv == lane8).astype(jnp.float32)
                rank = jnp.dot(ltri, onehot2,
                               preferred_element_type=jnp.float32)
                rank = rank + carry2_ref[...]
                rank2_ref[:, k:k + 1] = jnp.sum(
                    onehot2 * rank, axis=-1, keepdims=True)
                carry2_ref[...] += jnp.sum(onehot2, axis=0, keepdims=True)

    @pl.when(i == 11)
    def _():
        cnt2 = carry2_ref[...]
        padded = jnp.ceil(cnt2 / blk) * blk
        offv = jnp.dot(padded, u8, preferred_element_type=jnp.float32)
        off_ref[...] = offv
        tcol = jax.lax.broadcasted_iota(jnp.int32, (ntile, 1), 0)
        ge = (tcol.astype(jnp.float32) >= offv / blk).astype(jnp.int32)
        te = jnp.sum(ge, axis=-1, keepdims=True) - 1
        oec = offv + cnt2
        oe = jnp.sum((te == lane8).astype(jnp.float32) * oec,
                     axis=-1, keepdims=True)
        te_ref[...] = te
        oe_ref[...] = oe.astype(jnp.int32)

    @pl.when(i >= 12)
    def _():
        offv = off_ref[...]
        for k in range(4):
            @pl.when(i == k + 12)
            def _(k=k):
                tmv = tm_ref[k * tb:(k + 1) * tb, :]
                onehot2 = (tmv == lane8).astype(jnp.float32)
                off_tok = jnp.sum(onehot2 * offv, axis=-1, keepdims=True)
                any_e = jnp.sum(onehot2, axis=-1, keepdims=True) > 0.5
                slot = off_tok + rank2_ref[:, k:k + 1]
                slot_ref[0] = jnp.where(any_e, slot.astype(jnp.int32),
                                        nslot - 1)


def _dispatch_body(x_hbm, slot_hbm, xs_hbm, idx_v, rows_v, sem1, sem2, *,
                   nc, rw):
    wid = lax.axis_index("s") * nc + lax.axis_index("c")
    c1 = pltpu.async_copy(slot_hbm.at[wid], idx_v, sem1)
    c2 = pltpu.async_copy(x_hbm.at[pl.ds(wid * rw, rw)], rows_v, sem2)
    c1.wait()
    c2.wait()
    pltpu.async_copy(rows_v, xs_hbm.at[idx_v], sem1).wait()


def _combine_body(ys_hbm, slot_hbm, y0_hbm, idx_v, rows_v, sem, *, nc, rw):
    wid = lax.axis_index("s") * nc + lax.axis_index("c")
    pltpu.sync_copy(slot_hbm.at[wid], idx_v)
    pltpu.async_copy(ys_hbm.at[idx_v], rows_v, sem).wait()
    pltpu.sync_copy(rows_v, y0_hbm.at[pl.ds(wid * rw, rw)])


def _gffn_body(te_ref, oe_ref, x_ref, w1_ref, b1_ref, w2_ref, b2_ref, y_ref,
               *, blk):
    t = pl.program_id(0)
    x = x_ref[...].astype(jnp.bfloat16)
    w1 = w1_ref[0].astype(jnp.bfloat16)
    h = jnp.dot(x, w1, preferred_element_type=jnp.float32)
    h = jnp.maximum(h + b1_ref[0], 0.0)
    w2 = w2_ref[0].astype(jnp.bfloat16)
    y = jnp.dot(h.astype(jnp.bfloat16), w2,
                preferred_element_type=jnp.float32)
    y = y + b2_ref[0]
    row = jax.lax.broadcasted_iota(jnp.int32, y.shape, 0) + t * blk
    y_ref[...] = jnp.where(row < oe_ref[t], y, 0.0)


def kernel(x, Wg, bg, W1, b1, W2, b2):
    b, s, d = x.shape
    nexp = Wg.shape[1]
    total = b * s
    cap = int(total / nexp * 1.25)
    nt = 8
    tb = total // nt
    blk = 128
    nslot = s + nexp * blk
    ntile = nslot // blk
    xf = x.reshape(total, d)

    route = functools.partial(
        _route_body, nexp=nexp, tb=tb, cap=cap, total=total, blk=blk,
        nslot=nslot, ntile=ntile)
    slot4, te, oe, psum, loss = pl.pallas_call(
        route,
        grid=(2 * nt,),
        in_specs=[
            pl.BlockSpec((tb, d), lambda i: (jnp.minimum(i, 7), 0)),
            pl.BlockSpec((d, nexp), lambda i: (0, 0)),
            pl.BlockSpec((1, nexp), lambda i: (0, 0)),
        ],
        out_specs=[
            pl.BlockSpec((1, tb, 1),
                         lambda i: (jnp.maximum(i - 12, 0), 0, 0)),
            pl.BlockSpec((ntile, 1), lambda i: (0, 0)),
            pl.BlockSpec((ntile, 1), lambda i: (0, 0)),
            pl.BlockSpec((1, nexp), lambda i: (0, 0)),
            pl.BlockSpec((1, 1), lambda i: (0, 0)),
        ],
        out_shape=[
            jax.ShapeDtypeStruct((4, tb, 1), jnp.int32),
            jax.ShapeDtypeStruct((ntile, 1), jnp.int32),
            jax.ShapeDtypeStruct((ntile, 1), jnp.int32),
            jax.ShapeDtypeStruct((1, nexp), jnp.float32),
            jax.ShapeDtypeStruct((1, 1), jnp.float32),
        ],
        scratch_shapes=[
            pltpu.VMEM((1, nexp), jnp.float32),
            pltpu.VMEM((tb, 4), jnp.int32),
            pltpu.VMEM((1, nexp), jnp.float32),
            pltpu.VMEM((s, 1), jnp.int32),
            pltpu.VMEM((1, nexp), jnp.float32),
            pltpu.VMEM((tb, 4), jnp.float32),
            pltpu.VMEM((1, nexp), jnp.float32),
        ],
    )(xf, Wg, bg.reshape(1, nexp))

    info = plsc.get_sparse_core_info()
    nc, ns = info.num_cores, info.num_subcores
    nw = nc * ns
    rw = s // nw
    slot_w = slot4.reshape(nw, rw)
    mesh = plsc.VectorSubcoreMesh(core_axis_name="c", subcore_axis_name="s")

    xs = pl.kernel(
        functools.partial(_dispatch_body, nc=nc, rw=rw),
        out_type=jax.ShapeDtypeStruct((nslot, d), jnp.float32),
        mesh=mesh,
        scratch_types=[
            pltpu.VMEM((rw,), jnp.int32),
            pltpu.VMEM((rw, d), jnp.float32),
            pltpu.SemaphoreType.DMA,
            pltpu.SemaphoreType.DMA,
        ],
    )(x[0], slot_w)

    gffn = functools.partial(_gffn_body, blk=blk)
    ys = pl.pallas_call(
        gffn,
        grid_spec=pltpu.PrefetchScalarGridSpec(
            num_scalar_prefetch=2,
            grid=(ntile,),
            in_specs=[
                pl.BlockSpec((blk, d), lambda t, te_r, oe_r: (t, 0)),
                pl.BlockSpec((1, d, d),
                             lambda t, te_r, oe_r: (te_r[t], 0, 0)),
                pl.BlockSpec((1, 1, d),
                             lambda t, te_r, oe_r: (te_r[t], 0, 0)),
                pl.BlockSpec((1, d, d),
                             lambda t, te_r, oe_r: (te_r[t], 0, 0)),
                pl.BlockSpec((1, 1, d),
                             lambda t, te_r, oe_r: (te_r[t], 0, 0)),
            ],
            out_specs=pl.BlockSpec((blk, d), lambda t, te_r, oe_r: (t, 0)),
        ),
        out_shape=jax.ShapeDtypeStruct((nslot, d), jnp.float32),
    )(te.reshape(ntile), oe.reshape(ntile), xs, W1,
      b1.reshape(nexp, 1, d), W2, b2.reshape(nexp, 1, d))

    y0 = pl.kernel(
        functools.partial(_combine_body, nc=nc, rw=rw),
        out_type=jax.ShapeDtypeStruct((s, d), jnp.float32),
        mesh=mesh,
        scratch_types=[
            pltpu.VMEM((rw,), jnp.int32),
            pltpu.VMEM((rw, d), jnp.float32),
            pltpu.SemaphoreType.DMA,
        ],
    )(ys, slot_w)

    out = jnp.concatenate([y0[None], jnp.zeros_like(y0)[None]], axis=0)
    return out, loss[0, 0]
